# baseline (speedup 1.0000x reference)
import functools

import jax
import jax.numpy as jnp
from jax.experimental import pallas as pl
from jax.experimental.pallas import tpu as pltpu


def _chamfer_body(ti, x1_ref, x2t_ref, d1_ref, i1_ref, d2_ref):
    i = pl.program_id(1)
    a = x1_ref[0]
    bt = x2t_ref[0]
    a0 = a[:, 0:1]
    a1 = a[:, 1:2]
    a2 = a[:, 2:3]
    b0 = bt[0:1, :]
    b1 = bt[1:2, :]
    b2 = bt[2:3, :]
    inner = jnp.dot(a, bt, preferred_element_type=jnp.float32)
    sq1 = a0 * a0 + a1 * a1 + a2 * a2
    sq2 = b0 * b0 + b1 * b1 + b2 * b2
    dist = (sq1 + sq2) - 2.0 * inner

    sl = pl.ds(i * ti, ti)
    d1_ref[0, 0, sl] = jnp.min(dist, axis=1)
    i1_ref[0, 0, sl] = jnp.argmin(dist, axis=1).astype(jnp.int32)

    cmin = jnp.min(dist, axis=0)

    @pl.when(i == 0)
    def _init():
        d2_ref[0, 0] = cmin

    @pl.when(i > 0)
    def _acc():
        d2_ref[0, 0] = jnp.minimum(d2_ref[0, 0], cmin)


@functools.partial(jax.jit, static_argnames=("ti",))
def _chamfer(xyz1, xyz2, ti=512):
    B, N1, _ = xyz1.shape
    N2 = xyz2.shape[1]
    x2t = xyz2.transpose(0, 2, 1)
    ni = N1 // ti
    grid = (B, ni)
    dist1, idx1, dist2 = pl.pallas_call(
        functools.partial(_chamfer_body, ti),
        grid=grid,
        in_specs=[
            pl.BlockSpec((1, ti, 3), lambda b, i: (b, i, 0)),
            pl.BlockSpec((1, 3, N2), lambda b, i: (b, 0, 0)),
        ],
        out_specs=[
            pl.BlockSpec((1, 1, N1), lambda b, i: (b, 0, 0)),
            pl.BlockSpec((1, 1, N1), lambda b, i: (b, 0, 0)),
            pl.BlockSpec((1, 1, N2), lambda b, i: (b, 0, 0)),
        ],
        out_shape=[
            jax.ShapeDtypeStruct((B, 1, N1), jnp.float32),
            jax.ShapeDtypeStruct((B, 1, N1), jnp.int32),
            jax.ShapeDtypeStruct((B, 1, N2), jnp.float32),
        ],
        compiler_params=pltpu.CompilerParams(
            dimension_semantics=("parallel", "arbitrary"),
        ),
    )(xyz1, x2t)
    return dist1.reshape(B, N1), dist2.reshape(B, N2), idx1.reshape(B, N1)


def kernel(xyz1, xyz2):
    return _chamfer(xyz1, xyz2)


# speedup vs baseline: 1.1026x; 1.1026x over previous
import functools

import jax
import jax.numpy as jnp
from jax.experimental import pallas as pl
from jax.experimental.pallas import tpu as pltpu


def _chamfer_body(ti, x1_ref, x2t2_ref, d1_ref, i1_ref, d2_ref):
    i = pl.program_id(1)
    a = x1_ref[0]
    bt = x2t2_ref[0]
    a0 = a[:, 0:1]
    a1 = a[:, 1:2]
    a2 = a[:, 2:3]
    b0 = bt[0:1, :]
    b1 = bt[1:2, :]
    b2 = bt[2:3, :]
    sq1 = a0 * a0 + a1 * a1 + a2 * a2
    sq2 = (b0 * b0 + b1 * b1 + b2 * b2) * 0.25
    inner2 = jnp.dot(a, bt, preferred_element_type=jnp.float32)
    dist = (sq1 + sq2) - inner2

    W = 128
    nc = dist.shape[1] // W
    run_val = dist[:, 0:W]
    run_chunk = jnp.zeros((ti, W), jnp.float32)
    for c in range(1, nc):
        d_c = dist[:, c * W:(c + 1) * W]
        pred = d_c < run_val
        run_val = jnp.where(pred, d_c, run_val)
        run_chunk = jnp.where(pred, jnp.float32(c), run_chunk)

    rmin = jnp.min(run_val, axis=1)
    jl = jax.lax.broadcasted_iota(jnp.int32, (ti, W), 1).astype(jnp.float32)
    jfull = run_chunk * jnp.float32(W) + jl
    masked = jnp.where(run_val == rmin[:, None], jfull, jnp.float32(jnp.inf))

    sl = pl.ds(i * ti, ti)
    d1_ref[0, 0, sl] = rmin
    i1_ref[0, 0, sl] = jnp.min(masked, axis=1).astype(jnp.int32)

    cmin = jnp.min(dist, axis=0)

    @pl.when(i == 0)
    def _init():
        d2_ref[0, 0] = cmin

    @pl.when(i > 0)
    def _acc():
        d2_ref[0, 0] = jnp.minimum(d2_ref[0, 0], cmin)


@functools.partial(jax.jit, static_argnames=("ti",))
def _chamfer(xyz1, xyz2, ti=512):
    B, N1, _ = xyz1.shape
    N2 = xyz2.shape[1]
    x2t2 = (xyz2 * 2.0).transpose(0, 2, 1)
    ni = N1 // ti
    grid = (B, ni)
    dist1, idx1, dist2 = pl.pallas_call(
        functools.partial(_chamfer_body, ti),
        grid=grid,
        in_specs=[
            pl.BlockSpec((1, ti, 3), lambda b, i: (b, i, 0)),
            pl.BlockSpec((1, 3, N2), lambda b, i: (b, 0, 0)),
        ],
        out_specs=[
            pl.BlockSpec((1, 1, N1), lambda b, i: (b, 0, 0)),
            pl.BlockSpec((1, 1, N1), lambda b, i: (b, 0, 0)),
            pl.BlockSpec((1, 1, N2), lambda b, i: (b, 0, 0)),
        ],
        out_shape=[
            jax.ShapeDtypeStruct((B, 1, N1), jnp.float32),
            jax.ShapeDtypeStruct((B, 1, N1), jnp.int32),
            jax.ShapeDtypeStruct((B, 1, N2), jnp.float32),
        ],
        compiler_params=pltpu.CompilerParams(
            dimension_semantics=("parallel", "arbitrary"),
        ),
    )(xyz1, x2t2)
    return dist1.reshape(B, N1), dist2.reshape(B, N2), idx1.reshape(B, N1)


def kernel(xyz1, xyz2):
    return _chamfer(xyz1, xyz2)


# speedup vs baseline: 1.4619x; 1.3259x over previous
import functools

import jax
import jax.numpy as jnp
from jax.experimental import pallas as pl
from jax.experimental.pallas import tpu as pltpu


def _chamfer_body(ti, x1_ref, x2t2_ref, d1_ref, i1_ref, d2_ref):
    i = pl.program_id(1)
    a = x1_ref[0]
    bt = x2t2_ref[0]
    a0 = a[:, 0:1]
    a1 = a[:, 1:2]
    a2 = a[:, 2:3]
    b0 = bt[0:1, :]
    b1 = bt[1:2, :]
    b2 = bt[2:3, :]
    sq1 = a0 * a0 + a1 * a1 + a2 * a2
    sq2 = (b0 * b0 + b1 * b1 + b2 * b2) * 0.25
    inner2 = jnp.dot(a, bt, preferred_element_type=jnp.float32)
    dist = (sq1 + sq2) - inner2

    W = 128
    nc = dist.shape[1] // W
    run_val = dist[:, 0:W]
    run_chunk = jnp.zeros((ti, W), jnp.float32)
    for c in range(1, nc):
        d_c = dist[:, c * W:(c + 1) * W]
        pred = d_c < run_val
        run_val = jnp.where(pred, d_c, run_val)
        run_chunk = jnp.where(pred, jnp.float32(c), run_chunk)

    rv_t = run_val.T
    rc_t = run_chunk.T
    rmin = jnp.min(rv_t, axis=0)
    jl_t = jax.lax.broadcasted_iota(jnp.int32, (W, ti), 0).astype(jnp.float32)
    jfull_t = rc_t * jnp.float32(W) + jl_t
    masked_t = jnp.where(rv_t == rmin[None, :], jfull_t, jnp.float32(jnp.inf))

    sl = pl.ds(i * ti, ti)
    d1_ref[0, 0, sl] = rmin
    i1_ref[0, 0, sl] = jnp.min(masked_t, axis=0).astype(jnp.int32)

    cmin = jnp.min(dist, axis=0)

    @pl.when(i == 0)
    def _init():
        d2_ref[0, 0] = cmin

    @pl.when(i > 0)
    def _acc():
        d2_ref[0, 0] = jnp.minimum(d2_ref[0, 0], cmin)


@functools.partial(jax.jit, static_argnames=("ti",))
def _chamfer(xyz1, xyz2, ti=512):
    B, N1, _ = xyz1.shape
    N2 = xyz2.shape[1]
    x2t2 = (xyz2 * 2.0).transpose(0, 2, 1)
    ni = N1 // ti
    grid = (B, ni)
    dist1, idx1, dist2 = pl.pallas_call(
        functools.partial(_chamfer_body, ti),
        grid=grid,
        in_specs=[
            pl.BlockSpec((1, ti, 3), lambda b, i: (b, i, 0)),
            pl.BlockSpec((1, 3, N2), lambda b, i: (b, 0, 0)),
        ],
        out_specs=[
            pl.BlockSpec((1, 1, N1), lambda b, i: (b, 0, 0)),
            pl.BlockSpec((1, 1, N1), lambda b, i: (b, 0, 0)),
            pl.BlockSpec((1, 1, N2), lambda b, i: (b, 0, 0)),
        ],
        out_shape=[
            jax.ShapeDtypeStruct((B, 1, N1), jnp.float32),
            jax.ShapeDtypeStruct((B, 1, N1), jnp.int32),
            jax.ShapeDtypeStruct((B, 1, N2), jnp.float32),
        ],
        compiler_params=pltpu.CompilerParams(
            dimension_semantics=("parallel", "arbitrary"),
        ),
    )(xyz1, x2t2)
    return dist1.reshape(B, N1), dist2.reshape(B, N2), idx1.reshape(B, N1)


def kernel(xyz1, xyz2):
    return _chamfer(xyz1, xyz2)


# speedup vs baseline: 1.5949x; 1.0910x over previous
import functools

import jax
import jax.numpy as jnp
from jax.experimental import pallas as pl
from jax.experimental.pallas import tpu as pltpu


def _chamfer_body(ti, x1_ref, x2t2_ref, d1_ref, i1_ref, d2_ref):
    i = pl.program_id(1)
    a = x1_ref[0]
    bt = x2t2_ref[0]
    a0 = a[:, 0:1]
    a1 = a[:, 1:2]
    a2 = a[:, 2:3]
    b0 = bt[0:1, :]
    b1 = bt[1:2, :]
    b2 = bt[2:3, :]
    sq1 = a0 * a0 + a1 * a1 + a2 * a2
    sq2 = (b0 * b0 + b1 * b1 + b2 * b2) * 0.25
    inner2 = jnp.dot(a, bt, preferred_element_type=jnp.float32)
    dist = (sq1 + sq2) - inner2

    W = 128
    nc = dist.shape[1] // W
    run_val = dist[:, 0:W]
    run_chunk = jnp.zeros((ti, W), jnp.float32)
    for c in range(1, nc):
        d_c = dist[:, c * W:(c + 1) * W]
        pred = d_c < run_val
        run_val = jnp.where(pred, d_c, run_val)
        run_chunk = jnp.where(pred, jnp.float32(c), run_chunk)

    rv_t = run_val.T
    rc_t = run_chunk.T
    rmin = jnp.min(rv_t, axis=0)
    jl_t = jax.lax.broadcasted_iota(jnp.int32, (W, ti), 0).astype(jnp.float32)
    jfull_t = rc_t * jnp.float32(W) + jl_t
    masked_t = jnp.where(rv_t == rmin[None, :], jfull_t, jnp.float32(jnp.inf))

    sl = pl.ds(i * ti, ti)
    d1_ref[0, 0, sl] = rmin
    i1_ref[0, 0, sl] = jnp.min(masked_t, axis=0).astype(jnp.int32)

    cmin = jnp.min(dist, axis=0)

    @pl.when(i == 0)
    def _init():
        d2_ref[0, 0] = cmin

    @pl.when(i > 0)
    def _acc():
        d2_ref[0, 0] = jnp.minimum(d2_ref[0, 0], cmin)


@functools.partial(jax.jit, static_argnames=("ti",))
def _chamfer(xyz1, xyz2, ti=1024):
    B, N1, _ = xyz1.shape
    N2 = xyz2.shape[1]
    x2t2 = (xyz2 * 2.0).transpose(0, 2, 1)
    ni = N1 // ti
    grid = (B, ni)
    dist1, idx1, dist2 = pl.pallas_call(
        functools.partial(_chamfer_body, ti),
        grid=grid,
        in_specs=[
            pl.BlockSpec((1, ti, 3), lambda b, i: (b, i, 0)),
            pl.BlockSpec((1, 3, N2), lambda b, i: (b, 0, 0)),
        ],
        out_specs=[
            pl.BlockSpec((1, 1, N1), lambda b, i: (b, 0, 0)),
            pl.BlockSpec((1, 1, N1), lambda b, i: (b, 0, 0)),
            pl.BlockSpec((1, 1, N2), lambda b, i: (b, 0, 0)),
        ],
        out_shape=[
            jax.ShapeDtypeStruct((B, 1, N1), jnp.float32),
            jax.ShapeDtypeStruct((B, 1, N1), jnp.int32),
            jax.ShapeDtypeStruct((B, 1, N2), jnp.float32),
        ],
        compiler_params=pltpu.CompilerParams(
            dimension_semantics=("parallel", "arbitrary"),
        ),
    )(xyz1, x2t2)
    return dist1.reshape(B, N1), dist2.reshape(B, N2), idx1.reshape(B, N1)


def kernel(xyz1, xyz2):
    return _chamfer(xyz1, xyz2)


# speedup vs baseline: 1.6250x; 1.0189x over previous
import functools

import jax
import jax.numpy as jnp
from jax.experimental import pallas as pl
from jax.experimental.pallas import tpu as pltpu


def _chamfer_body(ti, x1_ref, x2t2_ref, d1_ref, i1_ref, d2_ref):
    i = pl.program_id(1)
    a = x1_ref[0]
    bt = x2t2_ref[0]
    a0 = a[:, 0:1]
    a1 = a[:, 1:2]
    a2 = a[:, 2:3]
    b0 = bt[0:1, :]
    b1 = bt[1:2, :]
    b2 = bt[2:3, :]
    sq1 = a0 * a0 + a1 * a1 + a2 * a2
    sq2 = (b0 * b0 + b1 * b1 + b2 * b2) * 0.25
    inner2 = jnp.dot(a, bt, preferred_element_type=jnp.float32)
    dist = (sq1 + sq2) - inner2

    W = 128
    nc = dist.shape[1] // W
    run_val = dist[:, 0:W]
    run_chunk = jnp.zeros((ti, W), jnp.float32)
    for c in range(1, nc):
        d_c = dist[:, c * W:(c + 1) * W]
        pred = d_c < run_val
        run_val = jnp.where(pred, d_c, run_val)
        run_chunk = jnp.where(pred, jnp.float32(c), run_chunk)

    rv_t = run_val.T
    rc_t = run_chunk.T
    rmin = jnp.min(rv_t, axis=0)
    jl_t = jax.lax.broadcasted_iota(jnp.int32, (W, ti), 0).astype(jnp.float32)
    jfull_t = rc_t * jnp.float32(W) + jl_t
    masked_t = jnp.where(rv_t == rmin[None, :], jfull_t, jnp.float32(jnp.inf))

    sl = pl.ds(i * ti, ti)
    d1_ref[0, 0, sl] = rmin
    i1_ref[0, 0, sl] = jnp.min(masked_t, axis=0).astype(jnp.int32)

    cmin = jnp.min(dist, axis=0)

    @pl.when(i == 0)
    def _init():
        d2_ref[0, 0] = cmin

    @pl.when(i > 0)
    def _acc():
        d2_ref[0, 0] = jnp.minimum(d2_ref[0, 0], cmin)


@functools.partial(jax.jit, static_argnames=("ti",))
def _chamfer(xyz1, xyz2, ti=2048):
    B, N1, _ = xyz1.shape
    N2 = xyz2.shape[1]
    x2t2 = (xyz2 * 2.0).transpose(0, 2, 1)
    ni = N1 // ti
    grid = (B, ni)
    dist1, idx1, dist2 = pl.pallas_call(
        functools.partial(_chamfer_body, ti),
        grid=grid,
        in_specs=[
            pl.BlockSpec((1, ti, 3), lambda b, i: (b, i, 0)),
            pl.BlockSpec((1, 3, N2), lambda b, i: (b, 0, 0)),
        ],
        out_specs=[
            pl.BlockSpec((1, 1, N1), lambda b, i: (b, 0, 0)),
            pl.BlockSpec((1, 1, N1), lambda b, i: (b, 0, 0)),
            pl.BlockSpec((1, 1, N2), lambda b, i: (b, 0, 0)),
        ],
        out_shape=[
            jax.ShapeDtypeStruct((B, 1, N1), jnp.float32),
            jax.ShapeDtypeStruct((B, 1, N1), jnp.int32),
            jax.ShapeDtypeStruct((B, 1, N2), jnp.float32),
        ],
        compiler_params=pltpu.CompilerParams(
            dimension_semantics=("parallel", "arbitrary"),
        ),
    )(xyz1, x2t2)
    return dist1.reshape(B, N1), dist2.reshape(B, N2), idx1.reshape(B, N1)


def kernel(xyz1, xyz2):
    return _chamfer(xyz1, xyz2)


# speedup vs baseline: 1.6264x; 1.0009x over previous
import functools

import jax
import jax.numpy as jnp
from jax.experimental import pallas as pl
from jax.experimental.pallas import tpu as pltpu


def _chamfer_body(ti, x1_ref, x2t2_ref, d1_ref, i1_ref, d2_ref):
    i = pl.program_id(1)
    a = x1_ref[0]
    bt = x2t2_ref[0]
    a0 = a[:, 0:1]
    a1 = a[:, 1:2]
    a2 = a[:, 2:3]
    b0 = bt[0:1, :]
    b1 = bt[1:2, :]
    b2 = bt[2:3, :]
    sq1 = a0 * a0 + a1 * a1 + a2 * a2
    sq2 = (b0 * b0 + b1 * b1 + b2 * b2) * 0.25
    inner2 = jnp.dot(a.astype(jnp.bfloat16), bt.astype(jnp.bfloat16),
                     preferred_element_type=jnp.float32)
    dist = (sq1 + sq2) - inner2

    W = 128
    nc = dist.shape[1] // W
    run_val = dist[:, 0:W]
    run_chunk = jnp.zeros((ti, W), jnp.float32)
    for c in range(1, nc):
        d_c = dist[:, c * W:(c + 1) * W]
        pred = d_c < run_val
        run_val = jnp.where(pred, d_c, run_val)
        run_chunk = jnp.where(pred, jnp.float32(c), run_chunk)

    rv_t = run_val.T
    rc_t = run_chunk.T
    rmin = jnp.min(rv_t, axis=0)
    jl_t = jax.lax.broadcasted_iota(jnp.int32, (W, ti), 0).astype(jnp.float32)
    jfull_t = rc_t * jnp.float32(W) + jl_t
    masked_t = jnp.where(rv_t == rmin[None, :], jfull_t, jnp.float32(jnp.inf))

    sl = pl.ds(i * ti, ti)
    d1_ref[0, 0, sl] = rmin
    i1_ref[0, 0, sl] = jnp.min(masked_t, axis=0).astype(jnp.int32)

    cmin = jnp.min(dist, axis=0)

    @pl.when(i == 0)
    def _init():
        d2_ref[0, 0] = cmin

    @pl.when(i > 0)
    def _acc():
        d2_ref[0, 0] = jnp.minimum(d2_ref[0, 0], cmin)


@functools.partial(jax.jit, static_argnames=("ti",))
def _chamfer(xyz1, xyz2, ti=2048):
    B, N1, _ = xyz1.shape
    N2 = xyz2.shape[1]
    x2t2 = (xyz2 * 2.0).transpose(0, 2, 1)
    ni = N1 // ti
    grid = (B, ni)
    dist1, idx1, dist2 = pl.pallas_call(
        functools.partial(_chamfer_body, ti),
        grid=grid,
        in_specs=[
            pl.BlockSpec((1, ti, 3), lambda b, i: (b, i, 0)),
            pl.BlockSpec((1, 3, N2), lambda b, i: (b, 0, 0)),
        ],
        out_specs=[
            pl.BlockSpec((1, 1, N1), lambda b, i: (b, 0, 0)),
            pl.BlockSpec((1, 1, N1), lambda b, i: (b, 0, 0)),
            pl.BlockSpec((1, 1, N2), lambda b, i: (b, 0, 0)),
        ],
        out_shape=[
            jax.ShapeDtypeStruct((B, 1, N1), jnp.float32),
            jax.ShapeDtypeStruct((B, 1, N1), jnp.int32),
            jax.ShapeDtypeStruct((B, 1, N2), jnp.float32),
        ],
        compiler_params=pltpu.CompilerParams(
            dimension_semantics=("parallel", "arbitrary"),
        ),
    )(xyz1, x2t2)
    return dist1.reshape(B, N1), dist2.reshape(B, N2), idx1.reshape(B, N1)


def kernel(xyz1, xyz2):
    return _chamfer(xyz1, xyz2)
